# baseline (speedup 1.0000x reference)
import functools

import jax
import jax.numpy as jnp
from jax import lax
from jax.experimental import pallas as pl
from jax.experimental.pallas import tpu as pltpu
from jax.experimental.pallas import tpu_sc as plsc

N = 10000
E = 320000
LATENT = 256
HALF = 128
NUM_CT = 20


_ROWS = 2000


def _enc_body(xc, w1, b1, w2, b2, w3, b3, out):
    h1 = jnp.maximum(jnp.dot(xc[...], w1[...], preferred_element_type=jnp.float32) + b1[...], 0.0)
    h2 = jnp.maximum(jnp.dot(h1, w2[...], preferred_element_type=jnp.float32) + b2[...], 0.0)
    out[...] = jnp.dot(h2, w3[...], preferred_element_type=jnp.float32) + b3[...]


def _encoder(xc, p):
    grid = N // _ROWS
    full = lambda shape: pl.BlockSpec(shape, lambda i: (0, 0))
    return pl.pallas_call(
        _enc_body,
        grid=(grid,),
        in_specs=[
            pl.BlockSpec((_ROWS, 130), lambda i: (i, 0)),
            full((130, 512)), full((1, 512)),
            full((512, 256)), full((1, 256)),
            full((256, LATENT)), full((1, LATENT)),
        ],
        out_specs=pl.BlockSpec((_ROWS, LATENT), lambda i: (i, 0)),
        out_shape=jax.ShapeDtypeStruct((N, LATENT), jnp.float32),
    )(xc, p["mlp_W1"], p["mlp_b1"].reshape(1, -1), p["mlp_W2"],
      p["mlp_b2"].reshape(1, -1), p["mlp_W3"], p["mlp_b3"].reshape(1, -1))


def _xlxr_body(h, wl, bl, wr, br, xlb, xrb, xl0, xl1):
    xl = jnp.dot(h[...], wl[...], preferred_element_type=jnp.float32) + bl[...]
    xr = jnp.dot(h[...], wr[...], preferred_element_type=jnp.float32) + br[...]
    xlb[...] = xl.astype(jnp.bfloat16)
    xrb[...] = xr.astype(jnp.bfloat16)
    xl0[...] = xl[:, :HALF]
    xl1[...] = xl[:, HALF:]


def _xlxr(h, bp):
    grid = N // _ROWS
    full = lambda shape: pl.BlockSpec(shape, lambda i: (0, 0))
    hspec = pl.BlockSpec((_ROWS, HALF), lambda i: (i, 0))
    fspec = pl.BlockSpec((_ROWS, LATENT), lambda i: (i, 0))
    hshape = jax.ShapeDtypeStruct((N, HALF), jnp.float32)
    bshape = jax.ShapeDtypeStruct((N, LATENT), jnp.bfloat16)
    return pl.pallas_call(
        _xlxr_body,
        grid=(grid,),
        in_specs=[
            pl.BlockSpec((_ROWS, LATENT), lambda i: (i, 0)),
            full((LATENT, LATENT)), full((1, LATENT)),
            full((LATENT, LATENT)), full((1, LATENT)),
        ],
        out_specs=[fspec, fspec, hspec, hspec],
        out_shape=[bshape, bshape, hshape, hshape],
    )(h, bp["Wl"], bp["bl"].reshape(1, -1), bp["Wr"], bp["br"].reshape(1, -1))


def _post_body(h, s0, s1, den, gb, wc, bc, g1, b1, wf1, bf1, wf2, bf2, g2, b2, out):
    S = jnp.concatenate([s0[...], s1[...]], axis=-1)
    osp = S / (den[...] + 1e-16) + gb[...]
    o = h[...] + jnp.dot(osp, wc[...], preferred_element_type=jnp.float32) + bc[...]
    m1 = jnp.mean(o, axis=0, keepdims=True)
    v1 = jnp.mean((o - m1) * (o - m1), axis=0, keepdims=True)
    o = (o - m1) * jax.lax.rsqrt(v1 + 1e-5) * g1[...] + b1[...]
    hh = jnp.maximum(jnp.dot(o, wf1[...], preferred_element_type=jnp.float32) + bf1[...], 0.0)
    ffn = jnp.dot(hh, wf2[...], preferred_element_type=jnp.float32) + bf2[...]
    o2 = o + ffn
    m2 = jnp.mean(o2, axis=0, keepdims=True)
    v2 = jnp.mean((o2 - m2) * (o2 - m2), axis=0, keepdims=True)
    out[...] = (o2 - m2) * jax.lax.rsqrt(v2 + 1e-5) * g2[...] + b2[...]


def _post(h, s0, s1, den, bp):
    return pl.pallas_call(
        _post_body,
        out_shape=jax.ShapeDtypeStruct((N, LATENT), jnp.float32),
    )(h, s0, s1, den.reshape(N, 1), bp["gat_bias"].reshape(1, -1),
      bp["Wc"], bp["bc"].reshape(1, -1),
      bp["bn1_g"].reshape(1, -1), bp["bn1_b"].reshape(1, -1),
      bp["Wf1"], bp["bf1"].reshape(1, -1), bp["Wf2"], bp["bf2"].reshape(1, -1),
      bp["bn2_g"].reshape(1, -1), bp["bn2_b"].reshape(1, -1))


def _dec_body(h, wd, bd, out):
    logits = jnp.dot(h[...], wd[...], preferred_element_type=jnp.float32) + bd[...]
    m = jnp.max(logits, axis=-1, keepdims=True)
    e = jnp.exp(logits - m)
    out[...] = e / jnp.sum(e, axis=-1, keepdims=True)


def _decoder(h, p):
    wd = jnp.pad(p["dec_W"], ((0, 0), (0, HALF - NUM_CT)))
    bd = jnp.pad(p["dec_b"], (0, HALF - NUM_CT), constant_values=-1e30)
    probs = pl.pallas_call(
        _dec_body,
        grid=(N // _ROWS,),
        in_specs=[
            pl.BlockSpec((_ROWS, LATENT), lambda i: (i, 0)),
            pl.BlockSpec((LATENT, HALF), lambda i: (0, 0)),
            pl.BlockSpec((1, HALF), lambda i: (0, 0)),
        ],
        out_specs=pl.BlockSpec((_ROWS, HALF), lambda i: (i, 0)),
        out_shape=jax.ShapeDtypeStruct((N, HALF), jnp.float32),
    )(h, wd, bd.reshape(1, -1))
    return probs[:, :NUM_CT]


_CH = 80
_EPT_A = E // 32
_EPT_B = E // 16


def _sc_mesh():
    return plsc.VectorSubcoreMesh(core_axis_name="c", subcore_axis_name="s")


_SC_PARAMS = pltpu.CompilerParams(needs_layout_passes=False)


_NCH_A = _EPT_A // _CH


def _pass_a(xl, xr, src, dst, ea, we, att):
    @functools.partial(
        pl.kernel,
        out_type=jax.ShapeDtypeStruct((E,), jnp.float32),
        mesh=_sc_mesh(),
        scratch_types=[
            pltpu.VMEM((_CH,), jnp.int32),
            pltpu.VMEM((_CH,), jnp.int32),
            pltpu.VMEM((_CH,), jnp.float32),
            pltpu.VMEM((_CH,), jnp.int32),
            pltpu.VMEM((_CH,), jnp.int32),
            pltpu.VMEM((_CH,), jnp.float32),
            pltpu.VMEM((_CH, HALF), jnp.int32),
            pltpu.VMEM((_CH, HALF), jnp.int32),
            pltpu.VMEM((_CH, HALF), jnp.int32),
            pltpu.VMEM((_CH, HALF), jnp.int32),
            pltpu.VMEM((_CH,), jnp.float32),
            pltpu.VMEM((LATENT,), jnp.float32),
            pltpu.VMEM((LATENT,), jnp.float32),
            pltpu.VMEM((16, 17), jnp.float32),
            pltpu.SemaphoreType.DMA,
            pltpu.SemaphoreType.DMA,
        ],
        compiler_params=_SC_PARAMS,
    )
    def k(xlh, xrh, srch, dsth, eah, weh, atth, exh,
          srcA, dstA, eaA, srcB, dstB, eaB, aA, bA, aB, bB,
          ex_v, we_v, att_v, accbuf, semA, semB):
        cid = lax.axis_index("c")
        sid = lax.axis_index("s")
        wid = sid * 2 + cid
        base = wid * _EPT_A
        pltpu.sync_copy(weh, we_v)
        pltpu.sync_copy(atth, att_v)
        WCH = [we_v[pl.ds(i * 16, 16)] for i in range(LATENT // 16)]
        ACH = [att_v[pl.ds(i * 16, 16)] for i in range(LATENT // 16)]

        def fire(c, sv, dv, ev, ar, br, sem):
            off = base + c * _CH
            pltpu.sync_copy(srch.at[pl.ds(off, _CH)], sv)
            pltpu.sync_copy(dsth.at[pl.ds(off, _CH)], dv)
            pltpu.sync_copy(eah.at[pl.ds(off, _CH)], ev)
            pltpu.async_copy(xlh.at[sv], ar, sem)
            pltpu.async_copy(xrh.at[dv], br, sem)

        def drain(sv, dv, ar, br, sem):
            pltpu.make_async_copy(xlh.at[sv], ar, sem).wait()
            pltpu.make_async_copy(xrh.at[dv], br, sem).wait()

        def compute(c, ev, ar, br):
            off = base + c * _CH

            def group(g, carry):
                gb = g * 16
                eag = ev[pl.ds(gb, 16)]
                for e in range(16):
                    row = gb + e
                    eab = jnp.broadcast_to(eag[e], (16,))
                    acc0 = jnp.zeros((16,), jnp.float32)
                    acc1 = jnp.zeros((16,), jnp.float32)
                    for cc in range(LATENT // 32):
                        sl = pl.ds(cc * 16, 16)
                        a_lo, a_hi = plsc.unpack(
                            plsc.bitcast(ar[row, sl], jnp.bfloat16),
                            format=plsc.PackFormat.INTERLEAVED)
                        b_lo, b_hi = plsc.unpack(
                            plsc.bitcast(br[row, sl], jnp.bfloat16),
                            format=plsc.PackFormat.INTERLEAVED)
                        m = a_lo + b_lo + eab * WCH[2 * cc]
                        acc0 = acc0 + jnp.maximum(m, 0.2 * m) * ACH[2 * cc]
                        m = a_hi + b_hi + eab * WCH[2 * cc + 1]
                        acc1 = acc1 + jnp.maximum(m, 0.2 * m) * ACH[2 * cc + 1]
                    accbuf[e, pl.ds(0, 16)] = acc0 + acc1
                e16 = lax.iota(jnp.int32, 16)
                tots = [jnp.zeros((16,), jnp.float32) for _ in range(4)]
                for c2 in range(16):
                    cb = jnp.zeros((16,), jnp.int32) + c2
                    tots[c2 % 4] = tots[c2 % 4] + plsc.load_gather(accbuf, [e16, cb])
                tot = (tots[0] + tots[1]) + (tots[2] + tots[3])
                ex_v[pl.ds(gb, 16)] = jnp.exp(tot)
                return carry

            lax.fori_loop(0, _CH // 16, group, 0)
            pltpu.sync_copy(ex_v, exh.at[pl.ds(off, _CH)])

        fire(0, srcA, dstA, eaA, aA, bA, semA)

        def body2(j, carry):
            c0 = 2 * j
            fire(c0 + 1, srcB, dstB, eaB, aB, bB, semB)
            drain(srcA, dstA, aA, bA, semA)
            compute(c0, eaA, aA, bA)

            @pl.when(c0 + 2 < _NCH_A)
            def _():
                fire(c0 + 2, srcA, dstA, eaA, aA, bA, semA)

            drain(srcB, dstB, aB, bB, semB)
            compute(c0 + 1, eaB, aB, bB)
            return carry

        lax.fori_loop(0, _NCH_A // 2, body2, 0)
        drain(srcA, dstA, aA, bA, semA)
        compute(_NCH_A - 1, eaA, aA, bA)

    return k(xl, xr, src, dst, ea, we, att)


_NCH_B = _EPT_B // _CH


def _pass_b(xl0, xl1, src, dst, ex):
    oshape = jax.ShapeDtypeStruct((N, HALF), jnp.float32)
    @functools.partial(
        pl.kernel,
        out_type=[oshape, oshape, jax.ShapeDtypeStruct((N,), jnp.float32)],
        mesh=_sc_mesh(),
        scratch_types=[
            pltpu.VMEM((_CH,), jnp.int32),
            pltpu.VMEM((_CH,), jnp.int32),
            pltpu.VMEM((_CH,), jnp.float32),
            pltpu.VMEM((_CH,), jnp.int32),
            pltpu.VMEM((_CH,), jnp.int32),
            pltpu.VMEM((_CH,), jnp.float32),
            pltpu.VMEM((_CH, HALF), jnp.float32),
            pltpu.VMEM((_CH, HALF), jnp.float32),
            pltpu.VMEM((80, HALF), jnp.float32),
            pltpu.VMEM((1008,), jnp.float32),
            pltpu.VMEM_SHARED((N, HALF), jnp.float32),
            pltpu.VMEM_SHARED((N,), jnp.float32),
            pltpu.SemaphoreType.DMA,
            pltpu.SemaphoreType.DMA,
            pltpu.SemaphoreType.DMA,
            pltpu.SemaphoreType.DMA,
        ],
        compiler_params=_SC_PARAMS,
    )
    def k(xl0h, xl1h, srch, dsth, exh, s0h, s1h, denh,
          srcA, dstA, exA, srcB, dstB, exB, rowsA, rowsB,
          zbuf, zden, s_sh, den_sh, semGA, semGB, semSA, semSB):
        cid = lax.axis_index("c")
        sid = lax.axis_index("s")
        zv = jnp.zeros((16,), jnp.float32)

        def zrow(i, c):
            for cc in range(HALF // 16):
                zbuf[i, pl.ds(cc * 16, 16)] = zv
            return c

        lax.fori_loop(0, 80, zrow, 0)

        def zel(i, c):
            zden[pl.ds(i * 16, 16)] = zv
            return c

        lax.fori_loop(0, 63, zel, 0)

        for q in range(8):
            bid = sid + q * 16

            @pl.when(bid < 125)
            def _():
                pltpu.sync_copy(zbuf, s_sh.at[pl.ds(bid * 80, 80)])

        @pl.when(sid < 10)
        def _():
            pltpu.sync_copy(zden.at[pl.ds(0, 1000)], den_sh.at[pl.ds(sid * 1000, 1000)])

        plsc.subcore_barrier()

        base = sid * _EPT_B

        def fireG(c, sv, dv, ev, rows, semG):
            off = base + c * _CH
            pltpu.sync_copy(srch.at[pl.ds(off, _CH)], sv)
            pltpu.sync_copy(dsth.at[pl.ds(off, _CH)], dv)
            pltpu.sync_copy(exh.at[pl.ds(off, _CH)], ev)

            @pl.when(cid == 0)
            def _():
                pltpu.async_copy(xl0h.at[sv], rows, semG)

            @pl.when(cid == 1)
            def _():
                pltpu.async_copy(xl1h.at[sv], rows, semG)

        def drainG(sv, rows, semG):
            pltpu.make_async_copy(xl0h.at[sv], rows, semG).wait()

        def scale(rows, ev):
            for g in range(_CH // 16):
                exg = ev[pl.ds(g * 16, 16)]
                for e in range(16):
                    row = g * 16 + e
                    s = jnp.broadcast_to(exg[e], (16,))
                    for cc in range(HALF // 16):
                        sl = pl.ds(cc * 16, 16)
                        rows[row, sl] = rows[row, sl] * s

        def fireS(dv, ev, rows, semS):
            pltpu.async_copy(rows, s_sh.at[dv], semS, add=True)

            @pl.when(cid == 0)
            def _():
                pltpu.async_copy(ev, den_sh.at[dv], semS, add=True)

        def drainS(dv, ev, rows, semS):
            pltpu.make_async_copy(rows, s_sh.at[dv], semS).wait()

            @pl.when(cid == 0)
            def _():
                pltpu.make_async_copy(ev, den_sh.at[dv], semS).wait()

        fireG(0, srcA, dstA, exA, rowsA, semGA)

        def body2(j, carry):
            c0 = 2 * j

            @pl.when(j > 0)
            def _():
                drainS(dstB, exB, rowsB, semSB)

            fireG(c0 + 1, srcB, dstB, exB, rowsB, semGB)
            drainG(srcA, rowsA, semGA)
            scale(rowsA, exA)
            fireS(dstA, exA, rowsA, semSA)
            drainG(srcB, rowsB, semGB)
            scale(rowsB, exB)
            fireS(dstB, exB, rowsB, semSB)
            drainS(dstA, exA, rowsA, semSA)

            @pl.when(c0 + 2 < _NCH_B)
            def _():
                fireG(c0 + 2, srcA, dstA, exA, rowsA, semGA)

            return carry

        lax.fori_loop(0, _NCH_B // 2, body2, 0)
        drainS(dstB, exB, rowsB, semSB)
        plsc.subcore_barrier()

        for q in range(8):
            bid = sid + q * 16

            @pl.when((bid < 125) & (cid == 0))
            def _():
                sl = pl.ds(bid * 80, 80)
                pltpu.sync_copy(s_sh.at[sl], zbuf)
                pltpu.sync_copy(zbuf, s0h.at[sl])

            @pl.when((bid < 125) & (cid == 1))
            def _():
                sl = pl.ds(bid * 80, 80)
                pltpu.sync_copy(s_sh.at[sl], zbuf)
                pltpu.sync_copy(zbuf, s1h.at[sl])

        @pl.when((cid == 0) & (sid < 10))
        def _():
            pltpu.sync_copy(den_sh.at[pl.ds(sid * 1000, 1000)],
                            zden.at[pl.ds(0, 1000)])
            pltpu.sync_copy(zden.at[pl.ds(0, 1000)],
                            denh.at[pl.ds(sid * 1000, 1000)])

    return k(xl0, xl1, src, dst, ex)


def kernel(x, pos, edge_attr, edge_index, params):
    src = edge_index[0]
    dst = edge_index[1]
    ea = edge_attr[:, 0]
    xc = jnp.concatenate([x, pos], axis=-1)
    h = _encoder(xc, params)
    for bp in params["blocks"]:
        xlb, xrb, xl0, xl1 = _xlxr(h, bp)
        wep = bp["We"][0].reshape(8, 16, 2).transpose(0, 2, 1).reshape(-1)
        atp = bp["att"].reshape(8, 16, 2).transpose(0, 2, 1).reshape(-1)
        xlb32 = jax.lax.bitcast_convert_type(
            xlb.reshape(N, HALF, 2), jnp.int32)
        xrb32 = jax.lax.bitcast_convert_type(
            xrb.reshape(N, HALF, 2), jnp.int32)
        ex = _pass_a(xlb32, xrb32, src, dst, ea, wep, atp)
        s0, s1, den = _pass_b(xl0, xl1, src, dst, ex)
        h = _post(h, s0, s1, den, bp)
    return _decoder(h, params)


# speedup vs baseline: 1.2665x; 1.2665x over previous
import functools

import jax
import jax.numpy as jnp
from jax import lax
from jax.experimental import pallas as pl
from jax.experimental.pallas import tpu as pltpu
from jax.experimental.pallas import tpu_sc as plsc

N = 10000
E = 320000
LATENT = 256
HALF = 128
NUM_CT = 20


_ROWS = 2000


def _enc_body(xc, w1, b1, w2, b2, w3, b3, out):
    h1 = jnp.maximum(jnp.dot(xc[...], w1[...], preferred_element_type=jnp.float32) + b1[...], 0.0)
    h2 = jnp.maximum(jnp.dot(h1, w2[...], preferred_element_type=jnp.float32) + b2[...], 0.0)
    out[...] = jnp.dot(h2, w3[...], preferred_element_type=jnp.float32) + b3[...]


def _encoder(xc, p):
    grid = N // _ROWS
    full = lambda shape: pl.BlockSpec(shape, lambda i: (0, 0))
    return pl.pallas_call(
        _enc_body,
        grid=(grid,),
        in_specs=[
            pl.BlockSpec((_ROWS, 130), lambda i: (i, 0)),
            full((130, 512)), full((1, 512)),
            full((512, 256)), full((1, 256)),
            full((256, LATENT)), full((1, LATENT)),
        ],
        out_specs=pl.BlockSpec((_ROWS, LATENT), lambda i: (i, 0)),
        out_shape=jax.ShapeDtypeStruct((N, LATENT), jnp.float32),
    )(xc, p["mlp_W1"], p["mlp_b1"].reshape(1, -1), p["mlp_W2"],
      p["mlp_b2"].reshape(1, -1), p["mlp_W3"], p["mlp_b3"].reshape(1, -1))


def _xlxr_body(h, wl, bl, wr, br, xlf, xrf, xl0, xl1):
    xl = jnp.dot(h[...], wl[...], preferred_element_type=jnp.float32) + bl[...]
    xr = jnp.dot(h[...], wr[...], preferred_element_type=jnp.float32) + br[...]
    xlf[...] = xl
    xrf[...] = xr
    xl0[...] = xl[:, :HALF]
    xl1[...] = xl[:, HALF:]


def _xlxr(h, bp):
    grid = N // _ROWS
    full = lambda shape: pl.BlockSpec(shape, lambda i: (0, 0))
    hspec = pl.BlockSpec((_ROWS, HALF), lambda i: (i, 0))
    fspec = pl.BlockSpec((_ROWS, LATENT), lambda i: (i, 0))
    hshape = jax.ShapeDtypeStruct((N, HALF), jnp.float32)
    bshape = jax.ShapeDtypeStruct((N, LATENT), jnp.float32)
    return pl.pallas_call(
        _xlxr_body,
        grid=(grid,),
        in_specs=[
            pl.BlockSpec((_ROWS, LATENT), lambda i: (i, 0)),
            full((LATENT, LATENT)), full((1, LATENT)),
            full((LATENT, LATENT)), full((1, LATENT)),
        ],
        out_specs=[fspec, fspec, hspec, hspec],
        out_shape=[bshape, bshape, hshape, hshape],
    )(h, bp["Wl"], bp["bl"].reshape(1, -1), bp["Wr"], bp["br"].reshape(1, -1))


def _post_body(h, s0, s1, den, gb, wc, bc, g1, b1, wf1, bf1, wf2, bf2, g2, b2, out):
    S = jnp.concatenate([s0[...], s1[...]], axis=-1)
    osp = S / (den[...] + 1e-16) + gb[...]
    o = h[...] + jnp.dot(osp, wc[...], preferred_element_type=jnp.float32) + bc[...]
    m1 = jnp.mean(o, axis=0, keepdims=True)
    v1 = jnp.mean((o - m1) * (o - m1), axis=0, keepdims=True)
    o = (o - m1) * jax.lax.rsqrt(v1 + 1e-5) * g1[...] + b1[...]
    hh = jnp.maximum(jnp.dot(o, wf1[...], preferred_element_type=jnp.float32) + bf1[...], 0.0)
    ffn = jnp.dot(hh, wf2[...], preferred_element_type=jnp.float32) + bf2[...]
    o2 = o + ffn
    m2 = jnp.mean(o2, axis=0, keepdims=True)
    v2 = jnp.mean((o2 - m2) * (o2 - m2), axis=0, keepdims=True)
    out[...] = (o2 - m2) * jax.lax.rsqrt(v2 + 1e-5) * g2[...] + b2[...]


def _post(h, s0, s1, den, bp):
    return pl.pallas_call(
        _post_body,
        out_shape=jax.ShapeDtypeStruct((N, LATENT), jnp.float32),
    )(h, s0, s1, den.reshape(N, 1), bp["gat_bias"].reshape(1, -1),
      bp["Wc"], bp["bc"].reshape(1, -1),
      bp["bn1_g"].reshape(1, -1), bp["bn1_b"].reshape(1, -1),
      bp["Wf1"], bp["bf1"].reshape(1, -1), bp["Wf2"], bp["bf2"].reshape(1, -1),
      bp["bn2_g"].reshape(1, -1), bp["bn2_b"].reshape(1, -1))


def _dec_body(h, wd, bd, out):
    logits = jnp.dot(h[...], wd[...], preferred_element_type=jnp.float32) + bd[...]
    m = jnp.max(logits, axis=-1, keepdims=True)
    e = jnp.exp(logits - m)
    out[...] = e / jnp.sum(e, axis=-1, keepdims=True)


def _decoder(h, p):
    wd = jnp.pad(p["dec_W"], ((0, 0), (0, HALF - NUM_CT)))
    bd = jnp.pad(p["dec_b"], (0, HALF - NUM_CT), constant_values=-1e30)
    probs = pl.pallas_call(
        _dec_body,
        grid=(N // _ROWS,),
        in_specs=[
            pl.BlockSpec((_ROWS, LATENT), lambda i: (i, 0)),
            pl.BlockSpec((LATENT, HALF), lambda i: (0, 0)),
            pl.BlockSpec((1, HALF), lambda i: (0, 0)),
        ],
        out_specs=pl.BlockSpec((_ROWS, HALF), lambda i: (i, 0)),
        out_shape=jax.ShapeDtypeStruct((N, HALF), jnp.float32),
    )(h, wd, bd.reshape(1, -1))
    return probs[:, :NUM_CT]


_CH = 80
_EPT_A = E // 32
_EPT_B = E // 16


def _sc_mesh():
    return plsc.VectorSubcoreMesh(core_axis_name="c", subcore_axis_name="s")


_SC_PARAMS = pltpu.CompilerParams(needs_layout_passes=False)


_NCH_A = _EPT_A // _CH


def _pass_a(xl, xr, src, dst, ea, we, att):
    @functools.partial(
        pl.kernel,
        out_type=jax.ShapeDtypeStruct((E,), jnp.float32),
        mesh=_sc_mesh(),
        scratch_types=[
            pltpu.VMEM((10 * _CH,), jnp.int32),
            pltpu.VMEM((10 * _CH,), jnp.int32),
            pltpu.VMEM((10 * _CH,), jnp.float32),
            pltpu.VMEM((_CH, LATENT), jnp.float32),
            pltpu.VMEM((_CH, LATENT), jnp.float32),
            pltpu.VMEM((_CH, LATENT), jnp.float32),
            pltpu.VMEM((_CH, LATENT), jnp.float32),
            pltpu.VMEM((_CH,), jnp.float32),
            pltpu.VMEM((LATENT,), jnp.float32),
            pltpu.VMEM((LATENT,), jnp.float32),
            pltpu.VMEM((16, 17), jnp.float32),
            pltpu.SemaphoreType.DMA,
            pltpu.SemaphoreType.DMA,
        ],
        compiler_params=_SC_PARAMS,
    )
    def k(xlh, xrh, srch, dsth, eah, weh, atth, exh,
          srcS, dstS, eaS, aA, bA, aB, bB,
          ex_v, we_v, att_v, accbuf, semA, semB):
        cid = lax.axis_index("c")
        sid = lax.axis_index("s")
        wid = sid * 2 + cid
        base = wid * _EPT_A
        pltpu.sync_copy(weh, we_v)
        pltpu.sync_copy(atth, att_v)
        WCH = [we_v[pl.ds(i * 16, 16)] for i in range(LATENT // 16)]
        ACH = [att_v[pl.ds(i * 16, 16)] for i in range(LATENT // 16)]
        SB = 5 * _CH

        def fire(c, ar, br, sem):
            s = c // 5
            q = c % 5
            slot = s % 2

            @pl.when(q == 0)
            def _():
                soff = base + s * SB
                pltpu.sync_copy(srch.at[pl.ds(soff, SB)],
                                srcS.at[pl.ds(slot * SB, SB)])
                pltpu.sync_copy(dsth.at[pl.ds(soff, SB)],
                                dstS.at[pl.ds(slot * SB, SB)])
                pltpu.sync_copy(eah.at[pl.ds(soff, SB)],
                                eaS.at[pl.ds(slot * SB, SB)])

            qo = slot * SB + q * _CH
            pltpu.async_copy(xlh.at[srcS.at[pl.ds(qo, _CH)]], ar, sem)
            pltpu.async_copy(xrh.at[dstS.at[pl.ds(qo, _CH)]], br, sem)

        def drain(ar, br, sem):
            dummy = srcS.at[pl.ds(0, _CH)]
            pltpu.make_async_copy(xlh.at[dummy], ar, sem).wait()
            pltpu.make_async_copy(xrh.at[dummy], br, sem).wait()

        def compute(c, ar, br):
            off = base + c * _CH
            s = c // 5
            q = c % 5
            slot = s % 2
            qo = slot * SB + q * _CH

            def group(g, carry):
                gb = g * 16
                eag = eaS[pl.ds(qo + gb, 16)]
                for e in range(16):
                    row = gb + e
                    eab = jnp.broadcast_to(eag[e], (16,))
                    acc0 = jnp.zeros((16,), jnp.float32)
                    acc1 = jnp.zeros((16,), jnp.float32)
                    for cc in range(LATENT // 16):
                        sl = pl.ds(cc * 16, 16)
                        m = ar[row, sl] + br[row, sl] + eab * WCH[cc]
                        t = jnp.maximum(m, 0.2 * m) * ACH[cc]
                        if cc % 2:
                            acc1 = acc1 + t
                        else:
                            acc0 = acc0 + t
                    accbuf[e, pl.ds(0, 16)] = acc0 + acc1
                e16 = lax.iota(jnp.int32, 16)
                tots = [jnp.zeros((16,), jnp.float32) for _ in range(4)]
                for c2 in range(16):
                    cb = jnp.zeros((16,), jnp.int32) + c2
                    tots[c2 % 4] = tots[c2 % 4] + plsc.load_gather(accbuf, [e16, cb])
                tot = (tots[0] + tots[1]) + (tots[2] + tots[3])
                ex_v[pl.ds(gb, 16)] = jnp.exp(tot)
                return carry

            lax.fori_loop(0, _CH // 16, group, 0)
            pltpu.sync_copy(ex_v, exh.at[pl.ds(off, _CH)])

        fire(0, aA, bA, semA)

        def body2(j, carry):
            c0 = 2 * j
            fire(c0 + 1, aB, bB, semB)
            drain(aA, bA, semA)
            compute(c0, aA, bA)

            @pl.when(c0 + 2 < _NCH_A)
            def _():
                fire(c0 + 2, aA, bA, semA)

            drain(aB, bB, semB)
            compute(c0 + 1, aB, bB)
            return carry

        lax.fori_loop(0, _NCH_A // 2, body2, 0)
        drain(aA, bA, semA)
        compute(_NCH_A - 1, aA, bA)

    return k(xl, xr, src, dst, ea, we, att)


_NCH_B = _EPT_B // _CH


def _pass_b(xl0, xl1, src, dst, ex):
    oshape = jax.ShapeDtypeStruct((N, HALF), jnp.float32)
    @functools.partial(
        pl.kernel,
        out_type=[oshape, oshape, jax.ShapeDtypeStruct((N,), jnp.float32)],
        mesh=_sc_mesh(),
        scratch_types=[
            pltpu.VMEM((_CH,), jnp.int32),
            pltpu.VMEM((_CH,), jnp.int32),
            pltpu.VMEM((_CH,), jnp.float32),
            pltpu.VMEM((_CH,), jnp.int32),
            pltpu.VMEM((_CH,), jnp.int32),
            pltpu.VMEM((_CH,), jnp.float32),
            pltpu.VMEM((_CH, HALF), jnp.float32),
            pltpu.VMEM((_CH, HALF), jnp.float32),
            pltpu.VMEM((80, HALF), jnp.float32),
            pltpu.VMEM((1008,), jnp.float32),
            pltpu.VMEM_SHARED((N, HALF), jnp.float32),
            pltpu.VMEM_SHARED((N,), jnp.float32),
            pltpu.SemaphoreType.DMA,
            pltpu.SemaphoreType.DMA,
            pltpu.SemaphoreType.DMA,
            pltpu.SemaphoreType.DMA,
        ],
        compiler_params=_SC_PARAMS,
    )
    def k(xl0h, xl1h, srch, dsth, exh, s0h, s1h, denh,
          srcA, dstA, exA, srcB, dstB, exB, rowsA, rowsB,
          zbuf, zden, s_sh, den_sh, semGA, semGB, semSA, semSB):
        cid = lax.axis_index("c")
        sid = lax.axis_index("s")
        zv = jnp.zeros((16,), jnp.float32)

        def zrow(i, c):
            for cc in range(HALF // 16):
                zbuf[i, pl.ds(cc * 16, 16)] = zv
            return c

        lax.fori_loop(0, 80, zrow, 0)

        def zel(i, c):
            zden[pl.ds(i * 16, 16)] = zv
            return c

        lax.fori_loop(0, 63, zel, 0)

        for q in range(8):
            bid = sid + q * 16

            @pl.when(bid < 125)
            def _():
                pltpu.sync_copy(zbuf, s_sh.at[pl.ds(bid * 80, 80)])

        @pl.when(sid < 10)
        def _():
            pltpu.sync_copy(zden.at[pl.ds(0, 1000)], den_sh.at[pl.ds(sid * 1000, 1000)])

        plsc.subcore_barrier()

        base = sid * _EPT_B

        def fireG(c, sv, dv, ev, rows, semG):
            off = base + c * _CH
            pltpu.sync_copy(srch.at[pl.ds(off, _CH)], sv)
            pltpu.sync_copy(dsth.at[pl.ds(off, _CH)], dv)
            pltpu.sync_copy(exh.at[pl.ds(off, _CH)], ev)

            @pl.when(cid == 0)
            def _():
                pltpu.async_copy(xl0h.at[sv], rows, semG)

            @pl.when(cid == 1)
            def _():
                pltpu.async_copy(xl1h.at[sv], rows, semG)

        def drainG(sv, rows, semG):
            pltpu.make_async_copy(xl0h.at[sv], rows, semG).wait()

        def scale(rows, ev):
            for g in range(_CH // 16):
                exg = ev[pl.ds(g * 16, 16)]
                for e in range(16):
                    row = g * 16 + e
                    s = jnp.broadcast_to(exg[e], (16,))
                    for cc in range(HALF // 16):
                        sl = pl.ds(cc * 16, 16)
                        rows[row, sl] = rows[row, sl] * s

        def fireS(dv, ev, rows, semS):
            pltpu.async_copy(rows, s_sh.at[dv], semS, add=True)

            @pl.when(cid == 0)
            def _():
                pltpu.async_copy(ev, den_sh.at[dv], semS, add=True)

        def drainS(dv, ev, rows, semS):
            pltpu.make_async_copy(rows, s_sh.at[dv], semS).wait()

            @pl.when(cid == 0)
            def _():
                pltpu.make_async_copy(ev, den_sh.at[dv], semS).wait()

        fireG(0, srcA, dstA, exA, rowsA, semGA)

        def body2(j, carry):
            c0 = 2 * j

            @pl.when(j > 0)
            def _():
                drainS(dstB, exB, rowsB, semSB)

            fireG(c0 + 1, srcB, dstB, exB, rowsB, semGB)
            drainG(srcA, rowsA, semGA)
            scale(rowsA, exA)
            fireS(dstA, exA, rowsA, semSA)
            drainG(srcB, rowsB, semGB)
            scale(rowsB, exB)
            fireS(dstB, exB, rowsB, semSB)
            drainS(dstA, exA, rowsA, semSA)

            @pl.when(c0 + 2 < _NCH_B)
            def _():
                fireG(c0 + 2, srcA, dstA, exA, rowsA, semGA)

            return carry

        lax.fori_loop(0, _NCH_B // 2, body2, 0)
        drainS(dstB, exB, rowsB, semSB)
        plsc.subcore_barrier()

        for q in range(8):
            bid = sid + q * 16

            @pl.when((bid < 125) & (cid == 0))
            def _():
                sl = pl.ds(bid * 80, 80)
                pltpu.sync_copy(s_sh.at[sl], zbuf)
                pltpu.sync_copy(zbuf, s0h.at[sl])

            @pl.when((bid < 125) & (cid == 1))
            def _():
                sl = pl.ds(bid * 80, 80)
                pltpu.sync_copy(s_sh.at[sl], zbuf)
                pltpu.sync_copy(zbuf, s1h.at[sl])

        @pl.when((cid == 0) & (sid < 10))
        def _():
            pltpu.sync_copy(den_sh.at[pl.ds(sid * 1000, 1000)],
                            zden.at[pl.ds(0, 1000)])
            pltpu.sync_copy(zden.at[pl.ds(0, 1000)],
                            denh.at[pl.ds(sid * 1000, 1000)])

    return k(xl0, xl1, src, dst, ex)


def kernel(x, pos, edge_attr, edge_index, params):
    src = edge_index[0]
    dst = edge_index[1]
    ea = edge_attr[:, 0]
    xc = jnp.concatenate([x, pos], axis=-1)
    h = _encoder(xc, params)
    for bp in params["blocks"]:
        xlf, xrf, xl0, xl1 = _xlxr(h, bp)
        ex = _pass_a(xlf, xrf, src, dst, ea, bp["We"][0], bp["att"])
        s0, s1, den = _pass_b(xl0, xl1, src, dst, ex)
        h = _post(h, s0, s1, den, bp)
    return _decoder(h, params)


# speedup vs baseline: 1.4401x; 1.1371x over previous
import functools

import jax
import jax.numpy as jnp
from jax import lax
from jax.experimental import pallas as pl
from jax.experimental.pallas import tpu as pltpu
from jax.experimental.pallas import tpu_sc as plsc

N = 10000
E = 320000
LATENT = 256
HALF = 128
NUM_CT = 20


_ROWS = 2000


def _enc_body(xc, w1, b1, w2, b2, w3, b3, out):
    h1 = jnp.maximum(jnp.dot(xc[...], w1[...], preferred_element_type=jnp.float32) + b1[...], 0.0)
    h2 = jnp.maximum(jnp.dot(h1, w2[...], preferred_element_type=jnp.float32) + b2[...], 0.0)
    out[...] = jnp.dot(h2, w3[...], preferred_element_type=jnp.float32) + b3[...]


def _encoder(xc, p):
    grid = N // _ROWS
    full = lambda shape: pl.BlockSpec(shape, lambda i: (0, 0))
    return pl.pallas_call(
        _enc_body,
        grid=(grid,),
        in_specs=[
            pl.BlockSpec((_ROWS, 130), lambda i: (i, 0)),
            full((130, 512)), full((1, 512)),
            full((512, 256)), full((1, 256)),
            full((256, LATENT)), full((1, LATENT)),
        ],
        out_specs=pl.BlockSpec((_ROWS, LATENT), lambda i: (i, 0)),
        out_shape=jax.ShapeDtypeStruct((N, LATENT), jnp.float32),
    )(xc, p["mlp_W1"], p["mlp_b1"].reshape(1, -1), p["mlp_W2"],
      p["mlp_b2"].reshape(1, -1), p["mlp_W3"], p["mlp_b3"].reshape(1, -1))


def _xlxr_body(h, wl, bl, wr, br, xlf, xrf, xl0, xl1):
    xl = jnp.dot(h[...], wl[...], preferred_element_type=jnp.float32) + bl[...]
    xr = jnp.dot(h[...], wr[...], preferred_element_type=jnp.float32) + br[...]
    xlf[...] = xl
    xrf[...] = xr
    xl0[...] = xl[:, :HALF]
    xl1[...] = xl[:, HALF:]


def _xlxr(h, bp):
    grid = N // _ROWS
    full = lambda shape: pl.BlockSpec(shape, lambda i: (0, 0))
    hspec = pl.BlockSpec((_ROWS, HALF), lambda i: (i, 0))
    fspec = pl.BlockSpec((_ROWS, LATENT), lambda i: (i, 0))
    hshape = jax.ShapeDtypeStruct((N, HALF), jnp.float32)
    bshape = jax.ShapeDtypeStruct((N, LATENT), jnp.float32)
    return pl.pallas_call(
        _xlxr_body,
        grid=(grid,),
        in_specs=[
            pl.BlockSpec((_ROWS, LATENT), lambda i: (i, 0)),
            full((LATENT, LATENT)), full((1, LATENT)),
            full((LATENT, LATENT)), full((1, LATENT)),
        ],
        out_specs=[fspec, fspec, hspec, hspec],
        out_shape=[bshape, bshape, hshape, hshape],
    )(h, bp["Wl"], bp["bl"].reshape(1, -1), bp["Wr"], bp["br"].reshape(1, -1))


def _post_body(h, s0, s1, den, gb, wc, bc, g1, b1, wf1, bf1, wf2, bf2, g2, b2, out):
    S = jnp.concatenate([s0[...], s1[...]], axis=-1)
    osp = S / (den[...] + 1e-16) + gb[...]
    o = h[...] + jnp.dot(osp, wc[...], preferred_element_type=jnp.float32) + bc[...]
    m1 = jnp.mean(o, axis=0, keepdims=True)
    v1 = jnp.mean((o - m1) * (o - m1), axis=0, keepdims=True)
    o = (o - m1) * jax.lax.rsqrt(v1 + 1e-5) * g1[...] + b1[...]
    hh = jnp.maximum(jnp.dot(o, wf1[...], preferred_element_type=jnp.float32) + bf1[...], 0.0)
    ffn = jnp.dot(hh, wf2[...], preferred_element_type=jnp.float32) + bf2[...]
    o2 = o + ffn
    m2 = jnp.mean(o2, axis=0, keepdims=True)
    v2 = jnp.mean((o2 - m2) * (o2 - m2), axis=0, keepdims=True)
    out[...] = (o2 - m2) * jax.lax.rsqrt(v2 + 1e-5) * g2[...] + b2[...]


def _post(h, s0, s1, den, bp):
    return pl.pallas_call(
        _post_body,
        out_shape=jax.ShapeDtypeStruct((N, LATENT), jnp.float32),
    )(h, s0, s1, den.reshape(N, 1), bp["gat_bias"].reshape(1, -1),
      bp["Wc"], bp["bc"].reshape(1, -1),
      bp["bn1_g"].reshape(1, -1), bp["bn1_b"].reshape(1, -1),
      bp["Wf1"], bp["bf1"].reshape(1, -1), bp["Wf2"], bp["bf2"].reshape(1, -1),
      bp["bn2_g"].reshape(1, -1), bp["bn2_b"].reshape(1, -1))


def _dec_body(h, wd, bd, out):
    logits = jnp.dot(h[...], wd[...], preferred_element_type=jnp.float32) + bd[...]
    m = jnp.max(logits, axis=-1, keepdims=True)
    e = jnp.exp(logits - m)
    out[...] = e / jnp.sum(e, axis=-1, keepdims=True)


def _decoder(h, p):
    wd = jnp.pad(p["dec_W"], ((0, 0), (0, HALF - NUM_CT)))
    bd = jnp.pad(p["dec_b"], (0, HALF - NUM_CT), constant_values=-1e30)
    probs = pl.pallas_call(
        _dec_body,
        grid=(N // _ROWS,),
        in_specs=[
            pl.BlockSpec((_ROWS, LATENT), lambda i: (i, 0)),
            pl.BlockSpec((LATENT, HALF), lambda i: (0, 0)),
            pl.BlockSpec((1, HALF), lambda i: (0, 0)),
        ],
        out_specs=pl.BlockSpec((_ROWS, HALF), lambda i: (i, 0)),
        out_shape=jax.ShapeDtypeStruct((N, HALF), jnp.float32),
    )(h, wd, bd.reshape(1, -1))
    return probs[:, :NUM_CT]


_CH = 80
_EPT_A = E // 32
_EPT_B = E // 16


def _sc_mesh():
    return plsc.VectorSubcoreMesh(core_axis_name="c", subcore_axis_name="s")


_SC_PARAMS = pltpu.CompilerParams(needs_layout_passes=False)


_NCH_A = _EPT_A // _CH


def _pass_a(xl, xr, src, dst, ea, we, att):
    @functools.partial(
        pl.kernel,
        out_type=jax.ShapeDtypeStruct((E,), jnp.float32),
        mesh=_sc_mesh(),
        scratch_types=[
            pltpu.VMEM((10 * _CH,), jnp.int32),
            pltpu.VMEM((10 * _CH,), jnp.int32),
            pltpu.VMEM((10 * _CH,), jnp.float32),
            pltpu.VMEM((_CH, LATENT), jnp.float32),
            pltpu.VMEM((_CH, LATENT), jnp.float32),
            pltpu.VMEM((_CH, LATENT), jnp.float32),
            pltpu.VMEM((_CH, LATENT), jnp.float32),
            pltpu.VMEM((_CH,), jnp.float32),
            pltpu.VMEM((LATENT,), jnp.float32),
            pltpu.VMEM((LATENT,), jnp.float32),
            pltpu.VMEM((16, 17), jnp.float32),
            pltpu.SemaphoreType.DMA,
            pltpu.SemaphoreType.DMA,
        ],
        compiler_params=_SC_PARAMS,
    )
    def k(xlh, xrh, srch, dsth, eah, weh, atth, exh,
          srcS, dstS, eaS, aA, bA, aB, bB,
          ex_v, we_v, att_v, accbuf, semA, semB):
        cid = lax.axis_index("c")
        sid = lax.axis_index("s")
        wid = sid * 2 + cid
        base = wid * _EPT_A
        pltpu.sync_copy(weh, we_v)
        pltpu.sync_copy(atth, att_v)
        WCH = [we_v[pl.ds(i * 16, 16)] for i in range(LATENT // 16)]
        ACH = [att_v[pl.ds(i * 16, 16)] for i in range(LATENT // 16)]
        SB = 5 * _CH

        def fire(c, ar, br, sem):
            s = c // 5
            q = c % 5
            slot = s % 2

            @pl.when(q == 0)
            def _():
                soff = base + s * SB
                pltpu.sync_copy(srch.at[pl.ds(soff, SB)],
                                srcS.at[pl.ds(slot * SB, SB)])
                pltpu.sync_copy(dsth.at[pl.ds(soff, SB)],
                                dstS.at[pl.ds(slot * SB, SB)])
                pltpu.sync_copy(eah.at[pl.ds(soff, SB)],
                                eaS.at[pl.ds(slot * SB, SB)])

            qo = slot * SB + q * _CH
            pltpu.async_copy(xlh.at[srcS.at[pl.ds(qo, _CH)]], ar, sem)
            pltpu.async_copy(xrh.at[dstS.at[pl.ds(qo, _CH)]], br, sem)

        def drain(ar, br, sem):
            dummy = srcS.at[pl.ds(0, _CH)]
            pltpu.make_async_copy(xlh.at[dummy], ar, sem).wait()
            pltpu.make_async_copy(xrh.at[dummy], br, sem).wait()

        def compute(c, ar, br):
            off = base + c * _CH
            s = c // 5
            q = c % 5
            slot = s % 2
            qo = slot * SB + q * _CH

            def group(g, carry):
                gb = g * 16
                eag = eaS[pl.ds(qo + gb, 16)]
                for e in range(16):
                    row = gb + e
                    eab = jnp.broadcast_to(eag[e], (16,))
                    acc0 = jnp.zeros((16,), jnp.float32)
                    acc1 = jnp.zeros((16,), jnp.float32)
                    for cc in range(LATENT // 16):
                        sl = pl.ds(cc * 16, 16)
                        m = ar[row, sl] + br[row, sl] + eab * WCH[cc]
                        t = jnp.maximum(m, 0.2 * m) * ACH[cc]
                        if cc % 2:
                            acc1 = acc1 + t
                        else:
                            acc0 = acc0 + t
                    accbuf[e, pl.ds(0, 16)] = acc0 + acc1
                e16 = lax.iota(jnp.int32, 16)
                tots = [jnp.zeros((16,), jnp.float32) for _ in range(4)]
                for c2 in range(16):
                    cb = jnp.zeros((16,), jnp.int32) + c2
                    tots[c2 % 4] = tots[c2 % 4] + plsc.load_gather(accbuf, [e16, cb])
                tot = (tots[0] + tots[1]) + (tots[2] + tots[3])
                ex_v[pl.ds(gb, 16)] = jnp.exp(tot)
                return carry

            lax.fori_loop(0, _CH // 16, group, 0)
            pltpu.sync_copy(ex_v, exh.at[pl.ds(off, _CH)])

        fire(0, aA, bA, semA)

        def body2(j, carry):
            c0 = 2 * j
            fire(c0 + 1, aB, bB, semB)
            drain(aA, bA, semA)
            compute(c0, aA, bA)

            @pl.when(c0 + 2 < _NCH_A)
            def _():
                fire(c0 + 2, aA, bA, semA)

            drain(aB, bB, semB)
            compute(c0 + 1, aB, bB)
            return carry

        lax.fori_loop(0, _NCH_A // 2, body2, 0)
        drain(aA, bA, semA)
        compute(_NCH_A - 1, aA, bA)

    return k(xl, xr, src, dst, ea, we, att)


_NCH_B = _EPT_B // _CH


def _pass_b(xl0, xl1, src, dst, ex):
    oshape = jax.ShapeDtypeStruct((N, HALF), jnp.float32)
    @functools.partial(
        pl.kernel,
        out_type=[oshape, oshape, jax.ShapeDtypeStruct((N,), jnp.float32)],
        mesh=_sc_mesh(),
        scratch_types=[
            pltpu.VMEM((10 * _CH,), jnp.int32),
            pltpu.VMEM((10 * _CH,), jnp.float32),
            pltpu.VMEM((_CH,), jnp.int32),
            pltpu.VMEM((_CH,), jnp.int32),
            pltpu.VMEM((_CH, HALF), jnp.float32),
            pltpu.VMEM((_CH, HALF), jnp.float32),
            pltpu.VMEM((80, HALF), jnp.float32),
            pltpu.VMEM((1008,), jnp.float32),
            pltpu.VMEM_SHARED((N, HALF), jnp.float32),
            pltpu.VMEM_SHARED((N,), jnp.float32),
            pltpu.SemaphoreType.DMA,
            pltpu.SemaphoreType.DMA,
            pltpu.SemaphoreType.DMA,
            pltpu.SemaphoreType.DMA,
        ],
        compiler_params=_SC_PARAMS,
    )
    def k(xl0h, xl1h, srch, dsth, exh, s0h, s1h, denh,
          srcS, exS, dstA, dstB, rowsA, rowsB,
          zbuf, zden, s_sh, den_sh, semGA, semGB, semSA, semSB):
        cid = lax.axis_index("c")
        sid = lax.axis_index("s")
        zv = jnp.zeros((16,), jnp.float32)

        def zrow(i, c):
            for cc in range(HALF // 16):
                zbuf[i, pl.ds(cc * 16, 16)] = zv
            return c

        lax.fori_loop(0, 80, zrow, 0)

        def zel(i, c):
            zden[pl.ds(i * 16, 16)] = zv
            return c

        lax.fori_loop(0, 63, zel, 0)

        for q in range(8):
            bid = sid + q * 16

            @pl.when(bid < 125)
            def _():
                pltpu.sync_copy(zbuf, s_sh.at[pl.ds(bid * 80, 80)])

        @pl.when(sid < 10)
        def _():
            pltpu.sync_copy(zden.at[pl.ds(0, 1000)], den_sh.at[pl.ds(sid * 1000, 1000)])

        plsc.subcore_barrier()

        base = sid * _EPT_B

        SB = 5 * _CH

        def fireG(c, dv, rows, semG):
            off = base + c * _CH
            s = c // 5
            q = c % 5
            slot = s % 2

            @pl.when(q == 0)
            def _():
                soff = base + s * SB
                pltpu.sync_copy(srch.at[pl.ds(soff, SB)],
                                srcS.at[pl.ds(slot * SB, SB)])
                pltpu.sync_copy(exh.at[pl.ds(soff, SB)],
                                exS.at[pl.ds(slot * SB, SB)])

            pltpu.sync_copy(dsth.at[pl.ds(off, _CH)], dv)
            sv = srcS.at[pl.ds(slot * SB + q * _CH, _CH)]

            @pl.when(cid == 0)
            def _():
                pltpu.async_copy(xl0h.at[sv], rows, semG)

            @pl.when(cid == 1)
            def _():
                pltpu.async_copy(xl1h.at[sv], rows, semG)

        def drainG(rows, semG):
            dummy = srcS.at[pl.ds(0, _CH)]
            pltpu.make_async_copy(xl0h.at[dummy], rows, semG).wait()

        def scale(rows, c):
            s = c // 5
            qo = (s % 2) * SB + (c % 5) * _CH
            for g in range(_CH // 16):
                exg = exS[pl.ds(qo + g * 16, 16)]
                for e in range(16):
                    row = g * 16 + e
                    s = jnp.broadcast_to(exg[e], (16,))
                    for cc in range(HALF // 16):
                        sl = pl.ds(cc * 16, 16)
                        rows[row, sl] = rows[row, sl] * s

        def fireS(c, dv, rows, semS):
            s = c // 5
            qo = (s % 2) * SB + (c % 5) * _CH
            pltpu.async_copy(rows, s_sh.at[dv], semS, add=True)

            @pl.when(cid == 0)
            def _():
                pltpu.async_copy(exS.at[pl.ds(qo, _CH)], den_sh.at[dv],
                                 semS, add=True)

        def drainS(dv, rows, semS):
            pltpu.make_async_copy(rows, s_sh.at[dv], semS).wait()

            @pl.when(cid == 0)
            def _():
                pltpu.make_async_copy(exS.at[pl.ds(0, _CH)], den_sh.at[dv],
                                      semS).wait()

        fireG(0, dstA, rowsA, semGA)

        def body2(j, carry):
            c0 = 2 * j

            @pl.when(j > 0)
            def _():
                drainS(dstB, rowsB, semSB)

            fireG(c0 + 1, dstB, rowsB, semGB)
            drainG(rowsA, semGA)
            scale(rowsA, c0)
            fireS(c0, dstA, rowsA, semSA)
            drainG(rowsB, semGB)
            scale(rowsB, c0 + 1)
            fireS(c0 + 1, dstB, rowsB, semSB)
            drainS(dstA, rowsA, semSA)

            @pl.when(c0 + 2 < _NCH_B)
            def _():
                fireG(c0 + 2, dstA, rowsA, semGA)

            return carry

        lax.fori_loop(0, _NCH_B // 2, body2, 0)
        drainS(dstB, rowsB, semSB)
        plsc.subcore_barrier()

        for q in range(8):
            bid = sid + q * 16

            @pl.when((bid < 125) & (cid == 0))
            def _():
                sl = pl.ds(bid * 80, 80)
                pltpu.sync_copy(s_sh.at[sl], zbuf)
                pltpu.sync_copy(zbuf, s0h.at[sl])

            @pl.when((bid < 125) & (cid == 1))
            def _():
                sl = pl.ds(bid * 80, 80)
                pltpu.sync_copy(s_sh.at[sl], zbuf)
                pltpu.sync_copy(zbuf, s1h.at[sl])

        @pl.when((cid == 0) & (sid < 10))
        def _():
            pltpu.sync_copy(den_sh.at[pl.ds(sid * 1000, 1000)],
                            zden.at[pl.ds(0, 1000)])
            pltpu.sync_copy(zden.at[pl.ds(0, 1000)],
                            denh.at[pl.ds(sid * 1000, 1000)])

    return k(xl0, xl1, src, dst, ex)


def kernel(x, pos, edge_attr, edge_index, params):
    src = edge_index[0]
    dst = edge_index[1]
    ea = edge_attr[:, 0]
    xc = jnp.concatenate([x, pos], axis=-1)
    h = _encoder(xc, params)
    for bp in params["blocks"]:
        xlf, xrf, xl0, xl1 = _xlxr(h, bp)
        ex = _pass_a(xlf, xrf, src, dst, ea, bp["We"][0], bp["att"])
        s0, s1, den = _pass_b(xl0, xl1, src, dst, ex)
        h = _post(h, s0, s1, den, bp)
    return _decoder(h, params)
